# baseline (speedup 1.0000x reference)
import functools

import jax
import jax.numpy as jnp
from jax import lax
from jax.experimental import pallas as pl
from jax.experimental.pallas import tpu as pltpu
from jax.experimental.pallas import tpu_sc as plsc

_PAD_IDX = 1
_BLK = 64


def _sc_gather(emb_weight, idx_flat, n_rows, d):
    info = plsc.get_sparse_core_info()
    nw = info.num_cores * info.num_subcores
    rows_per_w = n_rows // nw

    mesh = plsc.VectorSubcoreMesh(core_axis_name="c", subcore_axis_name="s")

    @functools.partial(
        pl.kernel,
        mesh=mesh,
        out_type=jax.ShapeDtypeStruct((n_rows, d), jnp.float32),
        scratch_types=[
            pltpu.VMEM((rows_per_w,), jnp.int32),
            pltpu.VMEM((rows_per_w, d), jnp.float32),
            pltpu.SemaphoreType.DMA,
        ],
    )
    def gather_kernel(emb_hbm, idx_hbm, out_hbm, idx_v, rows_v, sem):
        wid = lax.axis_index("s") * info.num_cores + lax.axis_index("c")
        base = wid * rows_per_w
        pltpu.sync_copy(idx_hbm.at[pl.ds(base, rows_per_w)], idx_v)
        pltpu.async_copy(emb_hbm.at[idx_v], rows_v, sem).wait()
        pltpu.sync_copy(rows_v, out_hbm.at[pl.ds(base, rows_per_w)])

    return gather_kernel(emb_weight, idx_flat)


def _merge_body(prompt_ref, x_ref, o_ref):
    j = pl.program_id(1)

    @pl.when(j == 0)
    def _():
        o_ref[...] = prompt_ref[...]

    @pl.when(j != 0)
    def _():
        o_ref[...] = x_ref[...]


def kernel(x, encoder_padding_mask, src_prompt, source_prompt_length_padding_mask, emb_weight):
    b, t, d = x.shape
    p = src_prompt.shape[1]

    idx = jnp.where(source_prompt_length_padding_mask, _PAD_IDX, src_prompt)
    idx_flat = idx.astype(jnp.int32).reshape(b * p)

    prompt_rows = _sc_gather(emb_weight, idx_flat, b * p, d)
    prompt_emb = prompt_rows.reshape(b, p, d)

    n_xblk = t // _BLK
    out = pl.pallas_call(
        _merge_body,
        grid=(b, 1 + n_xblk),
        in_specs=[
            pl.BlockSpec((1, _BLK, d), lambda bi, j: (bi, 0, 0)),
            pl.BlockSpec((1, _BLK, d), lambda bi, j: (bi, jnp.maximum(j - 1, 0), 0)),
        ],
        out_specs=pl.BlockSpec((1, _BLK, d), lambda bi, j: (bi, j, 0)),
        out_shape=jax.ShapeDtypeStruct((b, p + t, d), x.dtype),
        compiler_params=pltpu.CompilerParams(
            dimension_semantics=("parallel", "arbitrary"),
        ),
    )(prompt_emb, x)

    out_padding_mask = jnp.concatenate(
        [source_prompt_length_padding_mask, encoder_padding_mask], axis=1
    )
    return out, out_padding_mask
